# baseline (speedup 1.0000x reference)
import jax
import jax.numpy as jnp
from jax import lax
from jax.experimental import pallas as pl
from jax.experimental.pallas import tpu as pltpu
from jax.experimental.pallas import tpu_sc as plsc

DIM = 64
L = 50
K = 5
M = 10
B = 4096
LK = L - K
NEG_N = LK * M
ROWS = L + NEG_N
GC = 5
GW = ROWS // GC
VHALF = 500000

NUM_CORES = 2
NUM_SUBCORES = 16
NW = NUM_CORES * NUM_SUBCORES
WPB = B // NW
IDX_CHUNK = 16


def _dot_chunks(w_chunks, row_ref, r, off):
    acc = w_chunks[0] * row_ref[r, pl.ds(off, 16)]
    for c in range(1, 4):
        acc = acc + w_chunks[c] * row_ref[r, pl.ds(off + c * 16, 16)]
    return jnp.sum(acc)


def _sc_body(pidx_hbm, coff_hbm, emb_hbm, out_hbm,
             pidx_v, coff_v, rows_v, obuf_v, gsem, osem):
    cid = lax.axis_index("c")
    sid = lax.axis_index("s")
    wid = sid * NUM_CORES + cid
    base = wid * WPB

    def stage(chunk):
        pltpu.sync_copy(
            pidx_hbm.at[pl.ds(base + chunk * IDX_CHUNK, IDX_CHUNK)],
            pidx_v.at[chunk % 2])
        pltpu.sync_copy(
            coff_hbm.at[pl.ds(base + chunk * IDX_CHUNK, IDX_CHUNK)],
            coff_v.at[chunk % 2])

    def gather_descs(x):
        cpx = (x // IDX_CHUNK) % 2
        wj = x % IDX_CHUNK
        return [
            pltpu.make_async_copy(
                emb_hbm.at[pidx_v.at[cpx, wj, c]],
                rows_v.at[pl.ds(c * GW, GW)], gsem)
            for c in range(GC)
        ]

    lanes = lax.iota(jnp.int32, 16)

    def compute(x, obuf):
        cpx = (x // IDX_CHUNK) % 2
        wj = x % IDX_CHUNK

        def one_pos(i, carry):
            offs_w = coff_v[cpx, wj, pl.ds(i, 16)]
            offs_n = coff_v[cpx, wj, pl.ds(L + i * M, 16)]
            woff = offs_w[0]
            w = [rows_v[i, pl.ds(woff + c * 16, 16)] for c in range(4)]
            res = jnp.zeros((16,), jnp.float32)
            for k in range(K):
                s = _dot_chunks(w, rows_v, i + 1 + k, offs_w[1 + k])
                res = jnp.where(lanes == k, jnp.full((16,), s, jnp.float32), res)
            for m in range(M):
                s = _dot_chunks(w, rows_v, L + i * M + m, offs_n[m])
                res = jnp.where(lanes == K + m, jnp.full((16,), s, jnp.float32), res)
            obuf[i] = res
            return carry

        lax.fori_loop(0, LK, one_pos, 0)

    def slot(x, obuf):
        @pl.when(x % IDX_CHUNK == 0)
        def _():
            stage(x // IDX_CHUNK)

        for d in gather_descs(x):
            d.start()
        for d in gather_descs(x):
            d.wait()

        @pl.when(x >= 1)
        def _():
            pltpu.make_async_copy(obuf, out_hbm.at[base + x - 1], osem).wait()

        compute(x, obuf)
        pltpu.async_copy(obuf, out_hbm.at[base + x], osem)

    @pl.loop(0, WPB)
    def _iter(a):
        slot(a, obuf_v)

    pltpu.make_async_copy(obuf_v, out_hbm.at[base + WPB - 1], osem).wait()


@jax.jit
def _sc_call(pairidx, coloff, emb2):
    mesh = plsc.VectorSubcoreMesh(
        core_axis_name="c", subcore_axis_name="s",
        num_cores=NUM_CORES, num_subcores=NUM_SUBCORES)
    return pl.kernel(
        _sc_body,
        out_type=jax.ShapeDtypeStruct((B, LK, 16), jnp.float32),
        mesh=mesh,
        compiler_params=pltpu.CompilerParams(
            needs_layout_passes=False, use_tc_tiling_on_sc=False),
        scratch_types=[
            pltpu.VMEM((2, IDX_CHUNK, GC, GW), jnp.int32),
            pltpu.VMEM((2, IDX_CHUNK, 512), jnp.int32),
            pltpu.VMEM((ROWS, 2 * DIM), jnp.float32),
            pltpu.VMEM((LK, 16), jnp.float32),
            pltpu.SemaphoreType.DMA,
            pltpu.SemaphoreType.DMA,
        ],
    )(pairidx, coloff, emb2)


def kernel(walk, negative, emb):
    allidx = jnp.concatenate(
        [walk, negative.reshape(B, NEG_N)], axis=1)
    pairidx = (allidx >> 1).reshape(B, GC, GW)
    coloff = jnp.pad((allidx & 1) * DIM, ((0, 0), (0, 512 - ROWS)))
    emb2 = emb.reshape(VHALF, 2 * DIM)
    out = _sc_call(pairidx, coloff, emb2)
    pos = out[:, :, :K]
    neg = out[:, :, K:K + M]
    return pos, neg


# speedup vs baseline: 1.1891x; 1.1891x over previous
import jax
import jax.numpy as jnp
from jax import lax
from jax.experimental import pallas as pl
from jax.experimental.pallas import tpu as pltpu
from jax.experimental.pallas import tpu_sc as plsc

DIM = 64
L = 50
K = 5
M = 10
B = 4096
LK = L - K
NEG_N = LK * M
ROWS = L + NEG_N
GC = 5
GW = ROWS // GC

NUM_CORES = 2
NUM_SUBCORES = 16
NW = NUM_CORES * NUM_SUBCORES
WPB = B // NW
IDX_CHUNK = 16


def _dot(w_chunks, row_ref, r):
    acc = w_chunks[0] * row_ref[r, pl.ds(0, 16)]
    for c in range(1, 4):
        acc = acc + w_chunks[c] * row_ref[r, pl.ds(c * 16, 16)]
    return jnp.sum(acc)


def _sc_body(pidx_hbm, emb_hbm, out_hbm,
             pidx_v, rows_v, obuf_v, gsem, osem):
    cid = lax.axis_index("c")
    sid = lax.axis_index("s")
    wid = sid * NUM_CORES + cid
    base = wid * WPB

    def stage(chunk):
        pltpu.sync_copy(
            pidx_hbm.at[pl.ds(base + chunk * IDX_CHUNK, IDX_CHUNK)],
            pidx_v.at[chunk % 2])

    def gather_descs(x):
        cpx = (x // IDX_CHUNK) % 2
        wj = x % IDX_CHUNK
        return [
            pltpu.make_async_copy(
                emb_hbm.at[pidx_v.at[cpx, wj, c]],
                rows_v.at[pl.ds(c * GW, GW)], gsem)
            for c in range(GC)
        ]

    lanes = lax.iota(jnp.int32, 16)

    def compute(x, obuf):
        def one_pos(i, carry):
            w = [rows_v[i, pl.ds(c * 16, 16)] for c in range(4)]
            res = jnp.zeros((16,), jnp.float32)
            for k in range(K):
                s = _dot(w, rows_v, i + 1 + k)
                res = jnp.where(lanes == k, jnp.full((16,), s, jnp.float32), res)
            for m in range(M):
                s = _dot(w, rows_v, L + i * M + m)
                res = jnp.where(lanes == K + m, jnp.full((16,), s, jnp.float32), res)
            obuf[i] = res
            return carry

        lax.fori_loop(0, LK, one_pos, 0)

    def slot(x, obuf):
        @pl.when(x % IDX_CHUNK == 0)
        def _():
            stage(x // IDX_CHUNK)

        for d in gather_descs(x):
            d.start()
        for d in gather_descs(x):
            d.wait()

        @pl.when(x >= 1)
        def _():
            pltpu.make_async_copy(obuf, out_hbm.at[base + x - 1], osem).wait()

        compute(x, obuf)
        pltpu.async_copy(obuf, out_hbm.at[base + x], osem)

    @pl.loop(0, WPB)
    def _iter(a):
        slot(a, obuf_v)

    pltpu.make_async_copy(obuf_v, out_hbm.at[base + WPB - 1], osem).wait()


@jax.jit
def _sc_call(pidx, emb):
    mesh = plsc.VectorSubcoreMesh(
        core_axis_name="c", subcore_axis_name="s",
        num_cores=NUM_CORES, num_subcores=NUM_SUBCORES)
    return pl.kernel(
        _sc_body,
        out_type=jax.ShapeDtypeStruct((B, LK, 16), jnp.float32),
        mesh=mesh,
        compiler_params=pltpu.CompilerParams(
            needs_layout_passes=False, use_tc_tiling_on_sc=False),
        scratch_types=[
            pltpu.VMEM((2, IDX_CHUNK, GC, GW), jnp.int32),
            pltpu.VMEM((ROWS, DIM), jnp.float32),
            pltpu.VMEM((LK, 16), jnp.float32),
            pltpu.SemaphoreType.DMA,
            pltpu.SemaphoreType.DMA,
        ],
    )(pidx, emb)


def kernel(walk, negative, emb):
    allidx = jnp.concatenate(
        [walk, negative.reshape(B, NEG_N)], axis=1)
    pidx = allidx.reshape(B, GC, GW)
    out = _sc_call(pidx, emb)
    pos = out[:, :, :K]
    neg = out[:, :, K:K + M]
    return pos, neg


# speedup vs baseline: 1.4242x; 1.1977x over previous
import jax
import jax.numpy as jnp
from jax import lax
from jax.experimental import pallas as pl
from jax.experimental.pallas import tpu as pltpu
from jax.experimental.pallas import tpu_sc as plsc

DIM = 64
L = 50
K = 5
M = 10
B = 4096
LK = L - K
NEG_N = LK * M
ROWS = L + NEG_N
GC = 5
GW = ROWS // GC

NUM_CORES = 2
NUM_SUBCORES = 16
NW = NUM_CORES * NUM_SUBCORES
WPB = B // NW
IDX_CHUNK = 16


def _sc_body(pidx_hbm, emb_hbm, out_hbm,
             pidx_v, rows_v, obuf_v, gsem0, gsem1, osem):
    cid = lax.axis_index("c")
    sid = lax.axis_index("s")
    wid = sid * NUM_CORES + cid
    base = wid * WPB

    def stage(chunk):
        pltpu.sync_copy(
            pidx_hbm.at[pl.ds(base + chunk * IDX_CHUNK, IDX_CHUNK)],
            pidx_v.at[chunk % 2])

    def gather_descs(x, p, sem):
        cpx = (x // IDX_CHUNK) % 2
        wj = x % IDX_CHUNK
        return [
            pltpu.make_async_copy(
                emb_hbm.at[pidx_v.at[cpx, wj, c]],
                rows_v.at[p, pl.ds(c * GW, GW)], sem)
            for c in range(GC)
        ]

    lanes = lax.iota(jnp.int32, 16)

    def _dot(w_chunks, r, p):
        acc = w_chunks[0] * rows_v[p, r, pl.ds(0, 16)]
        for c in range(1, 4):
            acc = acc + w_chunks[c] * rows_v[p, r, pl.ds(c * 16, 16)]
        return jnp.sum(acc)

    def compute(x, p, obuf):
        def one_pos(i, carry):
            w = [rows_v[p, i, pl.ds(c * 16, 16)] for c in range(4)]
            res = jnp.zeros((16,), jnp.float32)
            for k in range(K):
                s = _dot(w, i + 1 + k, p)
                res = jnp.where(lanes == k, jnp.full((16,), s, jnp.float32), res)
            for m in range(M):
                s = _dot(w, L + i * M + m, p)
                res = jnp.where(lanes == K + m, jnp.full((16,), s, jnp.float32), res)
            obuf[i] = res
            return carry

        lax.fori_loop(0, LK, one_pos, 0)

    def slot(x, p):
        sem = gsem0 if p == 0 else gsem1
        nsem = gsem1 if p == 0 else gsem0

        for d in gather_descs(x, p, sem):
            d.wait()

        @pl.when(x + 1 < WPB)
        def _():
            @pl.when((x + 1) % IDX_CHUNK == 0)
            def _():
                stage((x + 1) // IDX_CHUNK)

            for d in gather_descs(x + 1, 1 - p, nsem):
                d.start()

        @pl.when(x >= 1)
        def _():
            pltpu.make_async_copy(obuf_v, out_hbm.at[base + x - 1], osem).wait()

        compute(x, p, obuf_v)
        pltpu.async_copy(obuf_v, out_hbm.at[base + x], osem)

    stage(0)
    for d in gather_descs(0, 0, gsem0):
        d.start()

    @pl.loop(0, WPB // 2)
    def _iter(h):
        slot(2 * h, 0)
        slot(2 * h + 1, 1)

    pltpu.make_async_copy(obuf_v, out_hbm.at[base + WPB - 1], osem).wait()


@jax.jit
def _sc_call(pidx, emb):
    mesh = plsc.VectorSubcoreMesh(
        core_axis_name="c", subcore_axis_name="s",
        num_cores=NUM_CORES, num_subcores=NUM_SUBCORES)
    return pl.kernel(
        _sc_body,
        out_type=jax.ShapeDtypeStruct((B, LK, 16), jnp.float32),
        mesh=mesh,
        compiler_params=pltpu.CompilerParams(
            needs_layout_passes=False, use_tc_tiling_on_sc=False),
        scratch_types=[
            pltpu.VMEM((2, IDX_CHUNK, GC, GW), jnp.int32),
            pltpu.VMEM((2, ROWS, DIM), jnp.float32),
            pltpu.VMEM((LK, 16), jnp.float32),
            pltpu.SemaphoreType.DMA,
            pltpu.SemaphoreType.DMA,
            pltpu.SemaphoreType.DMA,
        ],
    )(pidx, emb)


def kernel(walk, negative, emb):
    allidx = jnp.concatenate(
        [walk, negative.reshape(B, NEG_N)], axis=1)
    pidx = allidx.reshape(B, GC, GW)
    out = _sc_call(pidx, emb)
    pos = out[:, :, :K]
    neg = out[:, :, K:K + M]
    return pos, neg
